# baseline (speedup 1.0000x reference)
import functools

import jax
import jax.numpy as jnp
from jax import lax
from jax.experimental import pallas as pl
from jax.experimental.pallas import tpu as pltpu
from jax.experimental.pallas import tpu_sc as plsc

B = 1024
D = 128
N_CAND = 16384

NC = 2
NS = 16
NW = NC * NS

USERS_PER_W = B // NW
IDX_CHUNK = 128

N_HEAD = 3072
N_TAIL = N_CAND - N_HEAD
HEAD_PER_W = N_HEAD // NW
TAIL_PER_W = N_TAIL // NW

_MESH = plsc.VectorSubcoreMesh(core_axis_name="c", subcore_axis_name="s")


def _head_gather_body(qidx_hbm, iidx_hbm, user_hbm, item_hbm, u_out, it_out,
                      qi_v, u_v, ii_v, it_v, gsem, wsem):
    wid = lax.axis_index("s") * NC + lax.axis_index("c")

    ub = wid * USERS_PER_W
    pltpu.sync_copy(qidx_hbm.at[pl.ds(ub, USERS_PER_W)], qi_v)
    u_copy = pltpu.async_copy(user_hbm.at[qi_v], u_v, gsem)

    ib = wid * HEAD_PER_W
    pltpu.sync_copy(iidx_hbm.at[pl.ds(ib, HEAD_PER_W)], ii_v)
    it_copy = pltpu.async_copy(item_hbm.at[ii_v], it_v, gsem)

    u_copy.wait()
    u_wr = pltpu.async_copy(u_v, u_out.at[pl.ds(ub, USERS_PER_W)], wsem)
    it_copy.wait()
    it_wr = pltpu.async_copy(it_v, it_out.at[pl.ds(ib, HEAD_PER_W)], wsem)
    u_wr.wait()
    it_wr.wait()


_head_gather = functools.partial(
    pl.kernel,
    mesh=_MESH,
    out_type=[
        jax.ShapeDtypeStruct((B, D), jnp.float32),
        jax.ShapeDtypeStruct((N_HEAD, D), jnp.float32),
    ],
    scratch_types=[
        pltpu.VMEM((USERS_PER_W,), jnp.int32),
        pltpu.VMEM((USERS_PER_W, D), jnp.float32),
        pltpu.VMEM((HEAD_PER_W,), jnp.int32),
        pltpu.VMEM((HEAD_PER_W, D), jnp.float32),
        pltpu.SemaphoreType.DMA,
        pltpu.SemaphoreType.DMA,
    ],
)(_head_gather_body)


_TAIL_CHUNKS = []
_off = 0
while _off < TAIL_PER_W:
    _sz = min(IDX_CHUNK, TAIL_PER_W - _off)
    _TAIL_CHUNKS.append((_off, _sz))
    _off += _sz


def _tail_gather_body(iidx_hbm, item_hbm, it_out, ii_v, it_v, gsem, wsem):
    wid = lax.axis_index("s") * NC + lax.axis_index("c")
    ib = N_HEAD + wid * TAIL_PER_W
    pltpu.sync_copy(iidx_hbm.at[pl.ds(ib, TAIL_PER_W)], ii_v)
    gathers = []
    for off, sz in _TAIL_CHUNKS:
        gathers.append(pltpu.async_copy(
            item_hbm.at[ii_v.at[pl.ds(off, sz)]],
            it_v.at[pl.ds(off, sz)],
            gsem,
        ))
    writes = []
    for k, (off, sz) in enumerate(_TAIL_CHUNKS):
        gathers[k].wait()
        writes.append(pltpu.async_copy(
            it_v.at[pl.ds(off, sz)],
            it_out.at[pl.ds(wid * TAIL_PER_W + off, sz)],
            wsem,
        ))
    for w in writes:
        w.wait()


_tail_gather = functools.partial(
    pl.kernel,
    mesh=_MESH,
    out_type=jax.ShapeDtypeStruct((N_TAIL, D), jnp.float32),
    scratch_types=[
        pltpu.VMEM((TAIL_PER_W,), jnp.int32),
        pltpu.VMEM((TAIL_PER_W, D), jnp.float32),
        pltpu.SemaphoreType.DMA,
        pltpu.SemaphoreType.DMA,
    ],
)(_tail_gather_body)


TILE_HEAD = 1024
TILE_TAIL = 1024
HEAD_TILES = N_HEAD // TILE_HEAD
TAIL_TILES = N_TAIL // TILE_TAIL


def _mm_body(u_ref, it_ref, o_ref):
    o_ref[...] = lax.dot_general(
        u_ref[...], it_ref[...],
        dimension_numbers=(((1,), (1,)), ((), ())),
        preferred_element_type=jnp.float32,
    )


def _mm_acc_body(_, u_ref, it_ref, o_ref):
    o_ref[...] = lax.dot_general(
        u_ref[...], it_ref[...],
        dimension_numbers=(((1,), (1,)), ((), ())),
        preferred_element_type=jnp.float32,
    )


def kernel(support_indices, query_indices, item_idx, user_table, item_table):
    del support_indices
    qidx = query_indices.astype(jnp.int32)
    iidx = item_idx.astype(jnp.int32)

    u, it_head = _head_gather(qidx, iidx, user_table, item_table)
    it_tail = _tail_gather(iidx, item_table)

    scores = pl.pallas_call(
        _mm_body,
        grid=(HEAD_TILES,),
        in_specs=[
            pl.BlockSpec((B, D), lambda j: (0, 0)),
            pl.BlockSpec((TILE_HEAD, D), lambda j: (j, 0)),
        ],
        out_specs=pl.BlockSpec((B, TILE_HEAD), lambda j: (0, j)),
        out_shape=jax.ShapeDtypeStruct((B, N_CAND), jnp.float32),
    )(u, it_head)

    scores = pl.pallas_call(
        _mm_acc_body,
        grid=(TAIL_TILES,),
        in_specs=[
            pl.BlockSpec(memory_space=pl.MemorySpace.ANY),
            pl.BlockSpec((B, D), lambda j: (0, 0)),
            pl.BlockSpec((TILE_TAIL, D), lambda j: (j, 0)),
        ],
        out_specs=pl.BlockSpec(
            (B, TILE_TAIL), lambda j: (0, N_HEAD // TILE_TAIL + j)),
        out_shape=jax.ShapeDtypeStruct((B, N_CAND), jnp.float32),
        input_output_aliases={0: 0},
    )(scores, u, it_tail)
    return scores


# speedup vs baseline: 1.1171x; 1.1171x over previous
import functools

import jax
import jax.numpy as jnp
from jax import lax
from jax.experimental import pallas as pl
from jax.experimental.pallas import tpu as pltpu
from jax.experimental.pallas import tpu_sc as plsc

B = 1024
D = 128
N_CAND = 16384

NC = 2
NS = 16
NW = NC * NS

USERS_PER_W = B // NW
ITEMS_PER_W = N_CAND // NW
IDX_CHUNK = 128
N_CHUNKS = ITEMS_PER_W // IDX_CHUNK

_MESH = plsc.VectorSubcoreMesh(core_axis_name="c", subcore_axis_name="s")


def _gather_body(qidx_hbm, iidx_hbm, user_hbm, item_hbm, u_out, it_out,
                 qi_v, u_v, ii_v, it_v, gsem, wsem):
    wid = lax.axis_index("s") * NC + lax.axis_index("c")

    ub = wid * USERS_PER_W
    pltpu.sync_copy(qidx_hbm.at[pl.ds(ub, USERS_PER_W)], qi_v)
    u_copy = pltpu.async_copy(user_hbm.at[qi_v], u_v, gsem)

    ib = wid * ITEMS_PER_W
    pltpu.sync_copy(iidx_hbm.at[pl.ds(ib, ITEMS_PER_W)], ii_v)
    gathers = []
    for k in range(N_CHUNKS):
        gathers.append(pltpu.async_copy(
            item_hbm.at[ii_v.at[pl.ds(k * IDX_CHUNK, IDX_CHUNK)]],
            it_v.at[pl.ds(k * IDX_CHUNK, IDX_CHUNK)],
            gsem,
        ))

    u_copy.wait()
    u_wr = pltpu.async_copy(u_v, u_out.at[pl.ds(ub, USERS_PER_W)], wsem)

    writes = []
    for k in range(N_CHUNKS):
        gathers[k].wait()
        writes.append(pltpu.async_copy(
            it_v.at[pl.ds(k * IDX_CHUNK, IDX_CHUNK)],
            it_out.at[pl.ds(ib + k * IDX_CHUNK, IDX_CHUNK)],
            wsem,
        ))
    u_wr.wait()
    for w in writes:
        w.wait()


_gather = functools.partial(
    pl.kernel,
    mesh=_MESH,
    out_type=[
        jax.ShapeDtypeStruct((B, D), jnp.float32),
        jax.ShapeDtypeStruct((N_CAND, D), jnp.float32),
    ],
    scratch_types=[
        pltpu.VMEM((USERS_PER_W,), jnp.int32),
        pltpu.VMEM((USERS_PER_W, D), jnp.float32),
        pltpu.VMEM((ITEMS_PER_W,), jnp.int32),
        pltpu.VMEM((ITEMS_PER_W, D), jnp.float32),
        pltpu.SemaphoreType.DMA,
        pltpu.SemaphoreType.DMA,
    ],
)(_gather_body)


TILE_N = 2048


def _mm_body(u_ref, it_ref, o_ref):
    o_ref[...] = lax.dot_general(
        u_ref[...], it_ref[...],
        dimension_numbers=(((1,), (1,)), ((), ())),
        preferred_element_type=jnp.float32,
    )


def kernel(support_indices, query_indices, item_idx, user_table, item_table):
    del support_indices
    qidx = query_indices.astype(jnp.int32)
    iidx = item_idx.astype(jnp.int32)

    u, it = _gather(qidx, iidx, user_table, item_table)

    scores = pl.pallas_call(
        _mm_body,
        grid=(N_CAND // TILE_N,),
        in_specs=[
            pl.BlockSpec((B, D), lambda j: (0, 0)),
            pl.BlockSpec((TILE_N, D), lambda j: (j, 0)),
        ],
        out_specs=pl.BlockSpec((B, TILE_N), lambda j: (0, j)),
        out_shape=jax.ShapeDtypeStruct((B, N_CAND), jnp.float32),
    )(u, it)
    return scores


# speedup vs baseline: 1.1313x; 1.0126x over previous
import functools

import jax
import jax.numpy as jnp
from jax import lax
from jax.experimental import pallas as pl
from jax.experimental.pallas import tpu as pltpu
from jax.experimental.pallas import tpu_sc as plsc

B = 1024
D = 128
N_CAND = 16384

NC = 2
NS = 16
NW = NC * NS

USERS_PER_W = B // NW
ITEMS_PER_W = N_CAND // NW
IDX_CHUNK = 128
N_CHUNKS = ITEMS_PER_W // IDX_CHUNK

_MESH = plsc.VectorSubcoreMesh(core_axis_name="c", subcore_axis_name="s")


def _gather_body(qidx_hbm, iidx_hbm, user_hbm, item_hbm, u_out, it_out,
                 qi_v, u_v, ii_v, it_v, isem, usem, wsem, g0, g1, g2, g3):
    wid = lax.axis_index("s") * NC + lax.axis_index("c")
    ub = wid * USERS_PER_W
    ib = wid * ITEMS_PER_W
    gsems = [g0, g1, g2, g3]

    qi_copy = pltpu.async_copy(qidx_hbm.at[pl.ds(ub, USERS_PER_W)], qi_v, isem)
    ii_copy = pltpu.async_copy(iidx_hbm.at[pl.ds(ib, ITEMS_PER_W)], ii_v, usem)
    qi_copy.wait()
    ii_copy.wait()

    u_copy = pltpu.async_copy(user_hbm.at[qi_v], u_v, usem)

    gathers = []
    for k in range(N_CHUNKS):
        gathers.append(pltpu.async_copy(
            item_hbm.at[ii_v.at[pl.ds(k * IDX_CHUNK, IDX_CHUNK)]],
            it_v.at[pl.ds(k * IDX_CHUNK, IDX_CHUNK)],
            gsems[k],
        ))

    u_copy.wait()
    u_wr = pltpu.async_copy(u_v, u_out.at[pl.ds(ub, USERS_PER_W)], wsem)

    writes = []
    for k in range(N_CHUNKS):
        gathers[k].wait()
        writes.append(pltpu.async_copy(
            it_v.at[pl.ds(k * IDX_CHUNK, IDX_CHUNK)],
            it_out.at[pl.ds(ib + k * IDX_CHUNK, IDX_CHUNK)],
            wsem,
        ))
    u_wr.wait()
    for w in writes:
        w.wait()


_gather = functools.partial(
    pl.kernel,
    mesh=_MESH,
    out_type=[
        jax.ShapeDtypeStruct((B, D), jnp.float32),
        jax.ShapeDtypeStruct((N_CAND, D), jnp.float32),
    ],
    scratch_types=[
        pltpu.VMEM((USERS_PER_W,), jnp.int32),
        pltpu.VMEM((USERS_PER_W, D), jnp.float32),
        pltpu.VMEM((ITEMS_PER_W,), jnp.int32),
        pltpu.VMEM((ITEMS_PER_W, D), jnp.float32),
    ] + [pltpu.SemaphoreType.DMA] * (3 + N_CHUNKS),
)(_gather_body)


TILE_N = 2048


def _mm_body(u_ref, it_ref, o_ref):
    o_ref[...] = lax.dot_general(
        u_ref[...], it_ref[...],
        dimension_numbers=(((1,), (1,)), ((), ())),
        preferred_element_type=jnp.float32,
    )


def kernel(support_indices, query_indices, item_idx, user_table, item_table):
    del support_indices
    qidx = query_indices.astype(jnp.int32)
    iidx = item_idx.astype(jnp.int32)

    u, it = _gather(qidx, iidx, user_table, item_table)

    scores = pl.pallas_call(
        _mm_body,
        grid=(N_CAND // TILE_N,),
        in_specs=[
            pl.BlockSpec((B, D), lambda j: (0, 0)),
            pl.BlockSpec((TILE_N, D), lambda j: (j, 0)),
        ],
        out_specs=pl.BlockSpec((B, TILE_N), lambda j: (0, j)),
        out_shape=jax.ShapeDtypeStruct((B, N_CAND), jnp.float32),
    )(u, it)
    return scores
